# baseline (speedup 1.0000x reference)
import functools

import jax
import jax.numpy as jnp
from jax.experimental import pallas as pl
import jax.experimental.pallas.tpu as pltpu

L0 = 128
NH = 8
DH = L0 // NH
N_TOK = 32768
N_ENS = 16

BT = 1024
BQ = 512
BK = 1024
NQ = N_TOK // BQ
NKV = N_TOK // BK
INV_SCALE = 1.0 / (DH ** 0.5)
NEG = -1e30


def _layer_norm(x, g, b):
    m = jnp.mean(x, axis=-1, keepdims=True)
    v = jnp.mean((x - m) ** 2, axis=-1, keepdims=True)
    return (x - m) * jax.lax.rsqrt(v + 1e-5) * g + b


def _qkv_kernel(x_ref, g_ref, b_ref, w1_ref, w2_ref, b2_ref,
                wq_ref, bq_ref, wk_ref, bk_ref, wv_ref, bv_ref,
                qt_ref, ko_ref, vt_ref):
    x = x_ref[...]
    xn = _layer_norm(x, g_ref[...], b_ref[...])
    h = jnp.dot(xn, w1_ref[...], preferred_element_type=jnp.float32)
    h = h * jax.nn.sigmoid(h)
    kqv = jnp.dot(h, w2_ref[...], preferred_element_type=jnp.float32) + b2_ref[...]
    k = kqv[:, :L0]
    q = kqv[:, L0:2 * L0]
    v = kqv[:, 2 * L0:]
    qt_ref[...] = (jax.lax.dot_general(
        wq_ref[...], k, (((0,), (1,)), ((), ())),
        preferred_element_type=jnp.float32) + bq_ref[...]).astype(jnp.bfloat16)
    ko_ref[...] = (jnp.dot(q, wk_ref[...], preferred_element_type=jnp.float32)
                   + bk_ref[...]).astype(jnp.bfloat16)
    vt_ref[...] = (jax.lax.dot_general(
        wv_ref[...], v, (((0,), (1,)), ((), ())),
        preferred_element_type=jnp.float32) + bv_ref[...]).astype(jnp.bfloat16)


def _qkv_pass(x, p):
    nb = N_TOK // BT
    wspec = lambda shape: pl.BlockSpec(shape, lambda i: (0, 0))
    out_t = jax.ShapeDtypeStruct((L0, N_TOK), jnp.bfloat16)
    out_n = jax.ShapeDtypeStruct((N_TOK, L0), jnp.bfloat16)
    return pl.pallas_call(
        _qkv_kernel,
        grid=(nb,),
        in_specs=[
            pl.BlockSpec((BT, L0), lambda i: (i, 0)),
            wspec((1, L0)), wspec((1, L0)),
            wspec((L0, 4 * L0)), wspec((4 * L0, 3 * L0)), wspec((1, 3 * L0)),
            wspec((L0, L0)), wspec((L0, 1)),
            wspec((L0, L0)), wspec((1, L0)),
            wspec((L0, L0)), wspec((L0, 1)),
        ],
        out_specs=[pl.BlockSpec((L0, BT), lambda i: (0, i)),
                   pl.BlockSpec((BT, L0), lambda i: (i, 0)),
                   pl.BlockSpec((L0, BT), lambda i: (0, i))],
        out_shape=[out_t, out_n, out_t],
    )(x,
      p['kqv_ln_g'].reshape(1, L0), p['kqv_ln_b'].reshape(1, L0),
      p['kqv_w1'], p['kqv_w2'], p['kqv_b2'].reshape(1, 3 * L0),
      p['wq'] * INV_SCALE, p['bq'].reshape(L0, 1) * INV_SCALE,
      p['wk'], p['bk'].reshape(1, L0),
      p['wv'], p['bv'].reshape(L0, 1))


def _attn_kernel(kvlo_ref, kvcnt_ref, qelo_ref, qehi_ref, kelo_ref, kehi_ref,
                 qt_ref, k_ref, vt_ref, eq_ref, ek_ref, x_ref,
                 wo_ref, bo_ref, g_ref, b_ref, w1_ref, w2_ref, b2_ref,
                 *rest, final):
    if final:
        (fg_ref, fb_ref, fw1_ref, fw2_ref, fb2_ref, o_ref,
         m_s, l_s, acc_s) = rest
    else:
        (o_ref, m_s, l_s, acc_s) = rest
    i = pl.program_id(0)
    j = pl.program_id(1)

    @pl.when(j == 0)
    def _init():
        m_s[...] = jnp.full((NH, 1, BQ), NEG, jnp.float32)
        l_s[...] = jnp.zeros((NH, 1, BQ), jnp.float32)
        acc_s[...] = jnp.zeros((NH, DH, BQ), jnp.float32)

    def _flash(bias):
        ones_row = jnp.ones((1, BK), jnp.bfloat16)
        for h in range(NH):
            kh = k_ref[:, h * DH:(h + 1) * DH]
            qth = qt_ref[h * DH:(h + 1) * DH, :]
            vth = vt_ref[h * DH:(h + 1) * DH, :]
            st = jax.lax.dot_general(
                kh, qth, (((1,), (0,)), ((), ())),
                preferred_element_type=jnp.float32)
            if bias is not None:
                st = st + bias
            m_prev = m_s[h]
            m_new = jnp.maximum(m_prev, jnp.max(st, axis=0, keepdims=True))
            alpha = jnp.exp(m_prev - m_new)
            p = jnp.exp(st - m_new).astype(jnp.bfloat16)
            l_s[h] = l_s[h] * alpha + jax.lax.dot_general(
                ones_row, p, (((1,), (0,)), ((), ())),
                preferred_element_type=jnp.float32)
            acc_s[h] = acc_s[h] * alpha + jax.lax.dot_general(
                vth, p, (((1,), (0,)), ((), ())),
                preferred_element_type=jnp.float32)
            m_s[h] = m_new

    jj = jnp.minimum(kvlo_ref[i] + j, kvlo_ref[i] + kvcnt_ref[i] - 1)
    uniform = ((qelo_ref[i] == qehi_ref[i])
               & (kelo_ref[jj] == kehi_ref[jj])
               & (qelo_ref[i] == kelo_ref[jj]))
    active = j < kvcnt_ref[i]

    @pl.when(active & uniform)
    def _step_uniform():
        _flash(None)

    @pl.when(active & jnp.logical_not(uniform))
    def _step_masked():
        eq = eq_ref[0, 0, :]
        ek = ek_ref[0, 0, :]
        _flash(jnp.where(ek[:, None] == eq[None, :], 0.0, NEG))

    @pl.when(j == NKV - 1)
    def _epilogue():
        bf = jnp.bfloat16
        ot = jnp.concatenate(
            [acc_s[h] / l_s[h] for h in range(NH)], axis=0)
        o = ot.T
        attn = (jnp.dot(o.astype(bf), wo_ref[...].astype(bf),
                        preferred_element_type=jnp.float32)
                + bo_ref[...] + x_ref[...])
        xn = _layer_norm(attn, g_ref[...], b_ref[...])
        hh = jnp.dot(xn.astype(bf), w1_ref[...].astype(bf),
                     preferred_element_type=jnp.float32)
        hh = hh * jax.nn.sigmoid(hh)
        ff = (jnp.dot(hh.astype(bf), w2_ref[...].astype(bf),
                      preferred_element_type=jnp.float32) + b2_ref[...])
        res = ff + 2.0 * attn
        if final:
            xn2 = _layer_norm(res, fg_ref[...], fb_ref[...])
            h2 = jnp.dot(xn2, fw1_ref[...], preferred_element_type=jnp.float32)
            h2 = h2 * jax.nn.sigmoid(h2)
            o_ref[...] = (jnp.dot(h2, fw2_ref[...], preferred_element_type=jnp.float32)
                          + fb2_ref[...])
        else:
            o_ref[...] = res


def _attn_pass(qt, k, vt, eidx, x, p, ranges, fin=None):
    final = fin is not None
    eq = eidx.reshape(NQ, 1, BQ)
    ek = eidx.reshape(NKV, 1, BK)

    def kvmap(i, j, lo_ref, cnt_ref, *_):
        return (jnp.minimum(lo_ref[i] + j, lo_ref[i] + cnt_ref[i] - 1), 0)

    def kvmap_t(i, j, lo_ref, cnt_ref, *_):
        return (0, jnp.minimum(lo_ref[i] + j, lo_ref[i] + cnt_ref[i] - 1))

    def ekmap(i, j, lo_ref, cnt_ref, *_):
        return (jnp.minimum(lo_ref[i] + j, lo_ref[i] + cnt_ref[i] - 1), 0, 0)

    qmap = lambda i, j, *_: (i, 0)
    wmap = lambda i, j, *_: (0, 0)

    in_specs = [
        pl.BlockSpec((L0, BQ), lambda i, j, *_: (0, i)),
        pl.BlockSpec((BK, L0), kvmap),
        pl.BlockSpec((L0, BK), kvmap_t),
        pl.BlockSpec((1, 1, BQ), lambda i, j, *_: (i, 0, 0)),
        pl.BlockSpec((1, 1, BK), ekmap),
        pl.BlockSpec((BQ, L0), qmap),
        pl.BlockSpec((L0, L0), wmap), pl.BlockSpec((1, L0), wmap),
        pl.BlockSpec((1, L0), wmap), pl.BlockSpec((1, L0), wmap),
        pl.BlockSpec((L0, 4 * L0), wmap), pl.BlockSpec((4 * L0, L0), wmap),
        pl.BlockSpec((1, L0), wmap),
    ]
    args = [qt, k, vt, eq, ek, x,
            p['wo'], p['bo'].reshape(1, L0),
            p['ff_ln_g'].reshape(1, L0), p['ff_ln_b'].reshape(1, L0),
            p['ff_w1'], p['ff_w2'], p['ff_b2'].reshape(1, L0)]
    if final:
        in_specs += [
            pl.BlockSpec((1, L0), wmap), pl.BlockSpec((1, L0), wmap),
            pl.BlockSpec((L0, 4 * L0), wmap), pl.BlockSpec((4 * L0, 1), wmap),
            pl.BlockSpec((1, 1), wmap),
        ]
        args += [fin['ln_g'].reshape(1, L0), fin['ln_b'].reshape(1, L0),
                 fin['w1'], fin['w2'], fin['b2'].reshape(1, 1)]
        out_shape = jax.ShapeDtypeStruct((N_TOK, 1), jnp.float32)
        out_spec = pl.BlockSpec((BQ, 1), lambda i, j, *_: (i, 0))
    else:
        out_shape = jax.ShapeDtypeStruct((N_TOK, L0), jnp.float32)
        out_spec = pl.BlockSpec((BQ, L0), lambda i, j, *_: (i, 0))

    grid_spec = pltpu.PrefetchScalarGridSpec(
        num_scalar_prefetch=6,
        grid=(NQ, NKV),
        in_specs=in_specs,
        out_specs=out_spec,
        scratch_shapes=[
            pltpu.VMEM((NH, 1, BQ), jnp.float32),
            pltpu.VMEM((NH, 1, BQ), jnp.float32),
            pltpu.VMEM((NH, DH, BQ), jnp.float32),
        ],
    )
    return pl.pallas_call(
        functools.partial(_attn_kernel, final=final),
        grid_spec=grid_spec,
        out_shape=out_shape,
    )(*ranges, *args)


def _kv_ranges(eidx):
    e_q = eidx.reshape(NQ, BQ)
    e_k = eidx.reshape(NKV, BK)
    qe_lo = e_q[:, 0]
    qe_hi = e_q[:, -1]
    ke_lo = e_k[:, 0]
    ke_hi = e_k[:, -1]
    starts = jnp.searchsorted(eidx, jnp.arange(N_ENS, dtype=eidx.dtype),
                              side='left').astype(jnp.int32)
    ends = jnp.searchsorted(eidx, jnp.arange(N_ENS, dtype=eidx.dtype),
                            side='right').astype(jnp.int32)
    kv_lo = starts[qe_lo] // BK
    kv_hi = (ends[qe_hi] - 1) // BK
    return (kv_lo, kv_hi - kv_lo + 1, qe_lo, qe_hi, ke_lo, ke_hi)


def kernel(features, ensemble_index, p1, p2, fin):
    ranges = _kv_ranges(ensemble_index)
    qt1, k1, vt1 = _qkv_pass(features, p1)
    h1 = _attn_pass(qt1, k1, vt1, ensemble_index, features, p1, ranges)
    qt2, k2, vt2 = _qkv_pass(h1, p2)
    out = _attn_pass(qt2, k2, vt2, ensemble_index, h1, p2, ranges, fin=fin)
    return out


# speedup vs baseline: 1.0531x; 1.0531x over previous
import functools

import jax
import jax.numpy as jnp
from jax.experimental import pallas as pl
import jax.experimental.pallas.tpu as pltpu

L0 = 128
NH = 8
DH = L0 // NH
N_TOK = 32768
N_ENS = 16

BT = 1024
BQ = 512
BK = 1024
NQ = N_TOK // BQ
NKV = N_TOK // BK
INV_SCALE = 1.0 / (DH ** 0.5)
NEG = -1e30


def _layer_norm(x, g, b):
    m = jnp.mean(x, axis=-1, keepdims=True)
    v = jnp.mean((x - m) ** 2, axis=-1, keepdims=True)
    return (x - m) * jax.lax.rsqrt(v + 1e-5) * g + b


def _qkv_kernel(x_ref, g_ref, b_ref, w1_ref, w2_ref, b2_ref,
                wq_ref, bq_ref, wk_ref, bk_ref, wv_ref, bv_ref,
                qt_ref, ko_ref, vt_ref):
    x = x_ref[...]
    xn = _layer_norm(x, g_ref[...], b_ref[...])
    h = jnp.dot(xn, w1_ref[...], preferred_element_type=jnp.float32)
    h = h * jax.nn.sigmoid(h)
    kqv = jnp.dot(h, w2_ref[...], preferred_element_type=jnp.float32) + b2_ref[...]
    k = kqv[:, :L0]
    q = kqv[:, L0:2 * L0]
    v = kqv[:, 2 * L0:]
    qt_ref[...] = (jax.lax.dot_general(
        wq_ref[...], k, (((0,), (1,)), ((), ())),
        preferred_element_type=jnp.float32) + bq_ref[...]).astype(jnp.bfloat16)
    ko_ref[...] = (jnp.dot(q, wk_ref[...], preferred_element_type=jnp.float32)
                   + bk_ref[...]).astype(jnp.bfloat16)
    vt_ref[...] = (jax.lax.dot_general(
        wv_ref[...], v, (((0,), (1,)), ((), ())),
        preferred_element_type=jnp.float32) + bv_ref[...]).astype(jnp.bfloat16)


def _qkv_pass(x, p):
    nb = N_TOK // BT
    wspec = lambda shape: pl.BlockSpec(shape, lambda i: (0, 0))
    out_t = jax.ShapeDtypeStruct((L0, N_TOK), jnp.bfloat16)
    out_n = jax.ShapeDtypeStruct((N_TOK, L0), jnp.bfloat16)
    return pl.pallas_call(
        _qkv_kernel,
        grid=(nb,),
        in_specs=[
            pl.BlockSpec((BT, L0), lambda i: (i, 0)),
            wspec((1, L0)), wspec((1, L0)),
            wspec((L0, 4 * L0)), wspec((4 * L0, 3 * L0)), wspec((1, 3 * L0)),
            wspec((L0, L0)), wspec((L0, 1)),
            wspec((L0, L0)), wspec((1, L0)),
            wspec((L0, L0)), wspec((L0, 1)),
        ],
        out_specs=[pl.BlockSpec((L0, BT), lambda i: (0, i)),
                   pl.BlockSpec((BT, L0), lambda i: (i, 0)),
                   pl.BlockSpec((L0, BT), lambda i: (0, i))],
        out_shape=[out_t, out_n, out_t],
    )(x,
      p['kqv_ln_g'].reshape(1, L0), p['kqv_ln_b'].reshape(1, L0),
      p['kqv_w1'], p['kqv_w2'], p['kqv_b2'].reshape(1, 3 * L0),
      p['wq'] * INV_SCALE, p['bq'].reshape(L0, 1) * INV_SCALE,
      p['wk'], p['bk'].reshape(1, L0),
      p['wv'], p['bv'].reshape(L0, 1))


def _attn_kernel(kvlo_ref, kvcnt_ref, qelo_ref, qehi_ref, kelo_ref, kehi_ref,
                 qt_ref, k_ref, vt_ref, eq_ref, ek_ref, x_ref,
                 wo_ref, bo_ref, g_ref, b_ref, w1_ref, w2_ref, b2_ref,
                 *rest, final):
    if final:
        (fg_ref, fb_ref, fw1_ref, fw2_ref, fb2_ref, o_ref,
         m_s, l_s, acc_s) = rest
    else:
        (o_ref, m_s, l_s, acc_s) = rest
    i = pl.program_id(0)
    j = pl.program_id(1)

    @pl.when(j == 0)
    def _init():
        m_s[...] = jnp.full((NH, 1, BQ), NEG, jnp.float32)
        l_s[...] = jnp.zeros((NH, 1, BQ), jnp.float32)
        acc_s[...] = jnp.zeros((NH, DH, BQ), jnp.float32)

    def _flash(bias):
        ones_row = jnp.ones((1, BK), jnp.bfloat16)
        for h in range(NH):
            kh = k_ref[:, h * DH:(h + 1) * DH]
            qth = qt_ref[h * DH:(h + 1) * DH, :]
            vth = vt_ref[h * DH:(h + 1) * DH, :]
            st = jax.lax.dot_general(
                kh, qth, (((1,), (0,)), ((), ())),
                preferred_element_type=jnp.float32)
            if bias is not None:
                st = st + bias
            m_prev = m_s[h]
            m_new = jnp.maximum(m_prev, jnp.max(st, axis=0, keepdims=True))
            alpha = jnp.exp(m_prev - m_new)
            p = jnp.exp(st - m_new).astype(jnp.bfloat16)
            l_s[h] = l_s[h] * alpha + jax.lax.dot_general(
                ones_row, p, (((1,), (0,)), ((), ())),
                preferred_element_type=jnp.float32)
            acc_s[h] = acc_s[h] * alpha + jax.lax.dot_general(
                vth, p, (((1,), (0,)), ((), ())),
                preferred_element_type=jnp.float32)
            m_s[h] = m_new

    @pl.when(j < kvcnt_ref[i])
    def _step():
        eq = eq_ref[0, 0, :]
        ek = ek_ref[0, 0, :]
        _flash(jnp.where(ek[:, None] == eq[None, :], 0.0, NEG))

    @pl.when(j == NKV - 1)
    def _epilogue():
        bf = jnp.bfloat16
        ot = jnp.concatenate(
            [acc_s[h] / l_s[h] for h in range(NH)], axis=0)
        o = ot.T
        attn = (jnp.dot(o.astype(bf), wo_ref[...].astype(bf),
                        preferred_element_type=jnp.float32)
                + bo_ref[...] + x_ref[...])
        xn = _layer_norm(attn, g_ref[...], b_ref[...])
        hh = jnp.dot(xn.astype(bf), w1_ref[...].astype(bf),
                     preferred_element_type=jnp.float32)
        hh = hh * jax.nn.sigmoid(hh)
        ff = (jnp.dot(hh.astype(bf), w2_ref[...].astype(bf),
                      preferred_element_type=jnp.float32) + b2_ref[...])
        res = ff + 2.0 * attn
        if final:
            xn2 = _layer_norm(res, fg_ref[...], fb_ref[...])
            h2 = jnp.dot(xn2, fw1_ref[...], preferred_element_type=jnp.float32)
            h2 = h2 * jax.nn.sigmoid(h2)
            o_ref[...] = (jnp.dot(h2, fw2_ref[...], preferred_element_type=jnp.float32)
                          + fb2_ref[...])
        else:
            o_ref[...] = res


def _attn_pass(qt, k, vt, eidx, x, p, ranges, fin=None):
    final = fin is not None
    eq = eidx.reshape(NQ, 1, BQ)
    ek = eidx.reshape(NKV, 1, BK)

    def kvmap(i, j, lo_ref, cnt_ref, *_):
        return (jnp.minimum(lo_ref[i] + j, lo_ref[i] + cnt_ref[i] - 1), 0)

    def kvmap_t(i, j, lo_ref, cnt_ref, *_):
        return (0, jnp.minimum(lo_ref[i] + j, lo_ref[i] + cnt_ref[i] - 1))

    def ekmap(i, j, lo_ref, cnt_ref, *_):
        return (jnp.minimum(lo_ref[i] + j, lo_ref[i] + cnt_ref[i] - 1), 0, 0)

    qmap = lambda i, j, *_: (i, 0)
    wmap = lambda i, j, *_: (0, 0)

    in_specs = [
        pl.BlockSpec((L0, BQ), lambda i, j, *_: (0, i)),
        pl.BlockSpec((BK, L0), kvmap),
        pl.BlockSpec((L0, BK), kvmap_t),
        pl.BlockSpec((1, 1, BQ), lambda i, j, *_: (i, 0, 0)),
        pl.BlockSpec((1, 1, BK), ekmap),
        pl.BlockSpec((BQ, L0), qmap),
        pl.BlockSpec((L0, L0), wmap), pl.BlockSpec((1, L0), wmap),
        pl.BlockSpec((1, L0), wmap), pl.BlockSpec((1, L0), wmap),
        pl.BlockSpec((L0, 4 * L0), wmap), pl.BlockSpec((4 * L0, L0), wmap),
        pl.BlockSpec((1, L0), wmap),
    ]
    args = [qt, k, vt, eq, ek, x,
            p['wo'], p['bo'].reshape(1, L0),
            p['ff_ln_g'].reshape(1, L0), p['ff_ln_b'].reshape(1, L0),
            p['ff_w1'], p['ff_w2'], p['ff_b2'].reshape(1, L0)]
    if final:
        in_specs += [
            pl.BlockSpec((1, L0), wmap), pl.BlockSpec((1, L0), wmap),
            pl.BlockSpec((L0, 4 * L0), wmap), pl.BlockSpec((4 * L0, 1), wmap),
            pl.BlockSpec((1, 1), wmap),
        ]
        args += [fin['ln_g'].reshape(1, L0), fin['ln_b'].reshape(1, L0),
                 fin['w1'], fin['w2'], fin['b2'].reshape(1, 1)]
        out_shape = jax.ShapeDtypeStruct((N_TOK, 1), jnp.float32)
        out_spec = pl.BlockSpec((BQ, 1), lambda i, j, *_: (i, 0))
    else:
        out_shape = jax.ShapeDtypeStruct((N_TOK, L0), jnp.float32)
        out_spec = pl.BlockSpec((BQ, L0), lambda i, j, *_: (i, 0))

    grid_spec = pltpu.PrefetchScalarGridSpec(
        num_scalar_prefetch=6,
        grid=(NQ, NKV),
        in_specs=in_specs,
        out_specs=out_spec,
        scratch_shapes=[
            pltpu.VMEM((NH, 1, BQ), jnp.float32),
            pltpu.VMEM((NH, 1, BQ), jnp.float32),
            pltpu.VMEM((NH, DH, BQ), jnp.float32),
        ],
    )
    return pl.pallas_call(
        functools.partial(_attn_kernel, final=final),
        grid_spec=grid_spec,
        out_shape=out_shape,
    )(*ranges, *args)


def _kv_ranges(eidx):
    e_q = eidx.reshape(NQ, BQ)
    e_k = eidx.reshape(NKV, BK)
    qe_lo = e_q[:, 0]
    qe_hi = e_q[:, -1]
    ke_lo = e_k[:, 0]
    ke_hi = e_k[:, -1]
    starts = jnp.searchsorted(eidx, jnp.arange(N_ENS, dtype=eidx.dtype),
                              side='left').astype(jnp.int32)
    ends = jnp.searchsorted(eidx, jnp.arange(N_ENS, dtype=eidx.dtype),
                            side='right').astype(jnp.int32)
    kv_lo = starts[qe_lo] // BK
    kv_hi = (ends[qe_hi] - 1) // BK
    return (kv_lo, kv_hi - kv_lo + 1, qe_lo, qe_hi, ke_lo, ke_hi)


def kernel(features, ensemble_index, p1, p2, fin):
    ranges = _kv_ranges(ensemble_index)
    qt1, k1, vt1 = _qkv_pass(features, p1)
    h1 = _attn_pass(qt1, k1, vt1, ensemble_index, features, p1, ranges)
    qt2, k2, vt2 = _qkv_pass(h1, p2)
    out = _attn_pass(qt2, k2, vt2, ensemble_index, h1, p2, ranges, fin=fin)
    return out


# speedup vs baseline: 1.1880x; 1.1281x over previous
import functools

import jax
import jax.numpy as jnp
from jax.experimental import pallas as pl
import jax.experimental.pallas.tpu as pltpu

L0 = 128
NH = 8
DH = L0 // NH
N_TOK = 32768
N_ENS = 16

BT = 1024
BQ = 512
CH = 512
NQ = N_TOK // BQ
NCH = N_TOK // CH
INV_SCALE = 1.0 / (DH ** 0.5)
NEG = -1e30


def _layer_norm(x, g, b):
    m = jnp.mean(x, axis=-1, keepdims=True)
    v = jnp.mean((x - m) ** 2, axis=-1, keepdims=True)
    return (x - m) * jax.lax.rsqrt(v + 1e-5) * g + b


def _qkv_kernel(x_ref, g_ref, b_ref, w1_ref, w2_ref, b2_ref,
                wq_ref, bq_ref, wk_ref, bk_ref, wv_ref, bv_ref,
                qt_ref, ko_ref, vt_ref):
    x = x_ref[...]
    xn = _layer_norm(x, g_ref[...], b_ref[...])
    h = jnp.dot(xn, w1_ref[...], preferred_element_type=jnp.float32)
    h = h * jax.nn.sigmoid(h)
    kqv = jnp.dot(h, w2_ref[...], preferred_element_type=jnp.float32) + b2_ref[...]
    k = kqv[:, :L0]
    q = kqv[:, L0:2 * L0]
    v = kqv[:, 2 * L0:]
    qt_ref[...] = (jax.lax.dot_general(
        wq_ref[...], k, (((0,), (1,)), ((), ())),
        preferred_element_type=jnp.float32) + bq_ref[...]).astype(jnp.bfloat16)
    ko_ref[...] = (jnp.dot(q, wk_ref[...], preferred_element_type=jnp.float32)
                   + bk_ref[...]).astype(jnp.bfloat16)
    vt_ref[...] = (jax.lax.dot_general(
        wv_ref[...], v, (((0,), (1,)), ((), ())),
        preferred_element_type=jnp.float32) + bv_ref[...]).astype(jnp.bfloat16)


def _qkv_pass(x, p):
    nb = N_TOK // BT
    wspec = lambda shape: pl.BlockSpec(shape, lambda i: (0, 0))
    out_t = jax.ShapeDtypeStruct((L0, N_TOK), jnp.bfloat16)
    out_n = jax.ShapeDtypeStruct((N_TOK, L0), jnp.bfloat16)
    return pl.pallas_call(
        _qkv_kernel,
        grid=(nb,),
        in_specs=[
            pl.BlockSpec((BT, L0), lambda i: (i, 0)),
            wspec((1, L0)), wspec((1, L0)),
            wspec((L0, 4 * L0)), wspec((4 * L0, 3 * L0)), wspec((1, 3 * L0)),
            wspec((L0, L0)), wspec((L0, 1)),
            wspec((L0, L0)), wspec((1, L0)),
            wspec((L0, L0)), wspec((L0, 1)),
        ],
        out_specs=[pl.BlockSpec((L0, BT), lambda i: (0, i)),
                   pl.BlockSpec((BT, L0), lambda i: (i, 0)),
                   pl.BlockSpec((L0, BT), lambda i: (0, i))],
        out_shape=[out_t, out_n, out_t],
    )(x,
      p['kqv_ln_g'].reshape(1, L0), p['kqv_ln_b'].reshape(1, L0),
      p['kqv_w1'], p['kqv_w2'], p['kqv_b2'].reshape(1, 3 * L0),
      p['wq'] * INV_SCALE, p['bq'].reshape(L0, 1) * INV_SCALE,
      p['wk'], p['bk'].reshape(1, L0),
      p['wv'], p['bv'].reshape(L0, 1))


def _attn_kernel(chlo_ref, nch_ref,
                 qt_ref, k_ref, vt_ref, eq_ref, ea_ref, x_ref,
                 wo_ref, bo_ref, g_ref, b_ref, w1_ref, w2_ref, b2_ref,
                 *rest, final):
    if final:
        (fg_ref, fb_ref, fw1_ref, fw2_ref, fb2_ref, o_ref,
         m_s, l_s, acc_s) = rest
    else:
        (o_ref, m_s, l_s, acc_s) = rest
    i = pl.program_id(0)

    m_s[...] = jnp.full((NH, 1, BQ), NEG, jnp.float32)
    l_s[...] = jnp.zeros((NH, 1, BQ), jnp.float32)
    acc_s[...] = jnp.zeros((NH, DH, BQ), jnp.float32)

    eq = eq_ref[0, 0, :]
    ones_row = jnp.ones((1, CH), jnp.bfloat16)
    base = chlo_ref[i]

    def body(t, carry):
        start = (base + t) * CH
        ek = ea_ref[0, pl.ds(start, CH)]
        bias = jnp.where(ek[:, None] == eq[None, :], 0.0, NEG)
        for h in range(NH):
            kh = k_ref[pl.ds(start, CH), h * DH:(h + 1) * DH]
            qth = qt_ref[h * DH:(h + 1) * DH, :]
            vth = vt_ref[h * DH:(h + 1) * DH, pl.ds(start, CH)]
            st = jax.lax.dot_general(
                kh, qth, (((1,), (0,)), ((), ())),
                preferred_element_type=jnp.float32) + bias
            m_prev = m_s[h]
            m_new = jnp.maximum(m_prev, jnp.max(st, axis=0, keepdims=True))
            alpha = jnp.exp(m_prev - m_new)
            p = jnp.exp(st - m_new).astype(jnp.bfloat16)
            l_s[h] = l_s[h] * alpha + jax.lax.dot_general(
                ones_row, p, (((1,), (0,)), ((), ())),
                preferred_element_type=jnp.float32)
            acc_s[h] = acc_s[h] * alpha + jax.lax.dot_general(
                vth, p, (((1,), (0,)), ((), ())),
                preferred_element_type=jnp.float32)
            m_s[h] = m_new
        return carry

    jax.lax.fori_loop(0, nch_ref[i], body, 0)

    bf = jnp.bfloat16
    ot = jnp.concatenate(
        [acc_s[h] / l_s[h] for h in range(NH)], axis=0)
    o = ot.T
    attn = (jnp.dot(o.astype(bf), wo_ref[...].astype(bf),
                    preferred_element_type=jnp.float32)
            + bo_ref[...] + x_ref[...])
    xn = _layer_norm(attn, g_ref[...], b_ref[...])
    hh = jnp.dot(xn.astype(bf), w1_ref[...].astype(bf),
                 preferred_element_type=jnp.float32)
    hh = hh * jax.nn.sigmoid(hh)
    ff = (jnp.dot(hh.astype(bf), w2_ref[...].astype(bf),
                  preferred_element_type=jnp.float32) + b2_ref[...])
    res = ff + 2.0 * attn
    if final:
        xn2 = _layer_norm(res, fg_ref[...], fb_ref[...])
        h2 = jnp.dot(xn2, fw1_ref[...], preferred_element_type=jnp.float32)
        h2 = h2 * jax.nn.sigmoid(h2)
        o_ref[...] = (jnp.dot(h2, fw2_ref[...], preferred_element_type=jnp.float32)
                      + fb2_ref[...])
    else:
        o_ref[...] = res


def _attn_pass(qt, k, vt, eidx, x, p, ch_lo, nch, fin=None):
    final = fin is not None
    eq = eidx.reshape(NQ, 1, BQ)
    ea = eidx.reshape(1, N_TOK)

    qmap = lambda i, *_: (i, 0)
    wmap = lambda i, *_: (0, 0)

    in_specs = [
        pl.BlockSpec((L0, BQ), lambda i, *_: (0, i)),
        pl.BlockSpec((N_TOK, L0), wmap),
        pl.BlockSpec((L0, N_TOK), wmap),
        pl.BlockSpec((1, 1, BQ), lambda i, *_: (i, 0, 0)),
        pl.BlockSpec((1, N_TOK), wmap),
        pl.BlockSpec((BQ, L0), qmap),
        pl.BlockSpec((L0, L0), wmap), pl.BlockSpec((1, L0), wmap),
        pl.BlockSpec((1, L0), wmap), pl.BlockSpec((1, L0), wmap),
        pl.BlockSpec((L0, 4 * L0), wmap), pl.BlockSpec((4 * L0, L0), wmap),
        pl.BlockSpec((1, L0), wmap),
    ]
    args = [qt, k, vt, eq, ea, x,
            p['wo'], p['bo'].reshape(1, L0),
            p['ff_ln_g'].reshape(1, L0), p['ff_ln_b'].reshape(1, L0),
            p['ff_w1'], p['ff_w2'], p['ff_b2'].reshape(1, L0)]
    if final:
        in_specs += [
            pl.BlockSpec((1, L0), wmap), pl.BlockSpec((1, L0), wmap),
            pl.BlockSpec((L0, 4 * L0), wmap), pl.BlockSpec((4 * L0, 1), wmap),
            pl.BlockSpec((1, 1), wmap),
        ]
        args += [fin['ln_g'].reshape(1, L0), fin['ln_b'].reshape(1, L0),
                 fin['w1'], fin['w2'], fin['b2'].reshape(1, 1)]
        out_shape = jax.ShapeDtypeStruct((N_TOK, 1), jnp.float32)
        out_spec = pl.BlockSpec((BQ, 1), qmap)
    else:
        out_shape = jax.ShapeDtypeStruct((N_TOK, L0), jnp.float32)
        out_spec = pl.BlockSpec((BQ, L0), qmap)

    grid_spec = pltpu.PrefetchScalarGridSpec(
        num_scalar_prefetch=2,
        grid=(NQ,),
        in_specs=in_specs,
        out_specs=out_spec,
        scratch_shapes=[
            pltpu.VMEM((NH, 1, BQ), jnp.float32),
            pltpu.VMEM((NH, 1, BQ), jnp.float32),
            pltpu.VMEM((NH, DH, BQ), jnp.float32),
        ],
    )
    return pl.pallas_call(
        functools.partial(_attn_kernel, final=final),
        grid_spec=grid_spec,
        out_shape=out_shape,
    )(ch_lo, nch, *args)


def _kv_ranges(eidx):
    e_blk = eidx.reshape(NQ, BQ)
    e_lo = e_blk[:, 0]
    e_hi = e_blk[:, -1]
    starts = jnp.searchsorted(eidx, jnp.arange(N_ENS, dtype=eidx.dtype),
                              side='left').astype(jnp.int32)
    ends = jnp.searchsorted(eidx, jnp.arange(N_ENS, dtype=eidx.dtype),
                            side='right').astype(jnp.int32)
    ch_lo = starts[e_lo] // CH
    ch_hi = (ends[e_hi] - 1) // CH
    return ch_lo, ch_hi - ch_lo + 1


def kernel(features, ensemble_index, p1, p2, fin):
    ch_lo, nch = _kv_ranges(ensemble_index)
    qt1, k1, vt1 = _qkv_pass(features, p1)
    h1 = _attn_pass(qt1, k1, vt1, ensemble_index, features, p1, ch_lo, nch)
    qt2, k2, vt2 = _qkv_pass(h1, p2)
    out = _attn_pass(qt2, k2, vt2, ensemble_index, h1, p2, ch_lo, nch, fin=fin)
    return out


# speedup vs baseline: 1.1977x; 1.0082x over previous
import functools

import jax
import jax.numpy as jnp
from jax.experimental import pallas as pl
import jax.experimental.pallas.tpu as pltpu

L0 = 128
NH = 8
DH = L0 // NH
N_TOK = 32768
N_ENS = 16

BT = 1024
BQ = 512
CH = 1024
NQ = N_TOK // BQ
NCH = N_TOK // CH
INV_SCALE = 1.0 / (DH ** 0.5)
NEG = -1e30


def _layer_norm(x, g, b):
    m = jnp.mean(x, axis=-1, keepdims=True)
    v = jnp.mean((x - m) ** 2, axis=-1, keepdims=True)
    return (x - m) * jax.lax.rsqrt(v + 1e-5) * g + b


def _qkv_kernel(x_ref, g_ref, b_ref, w1_ref, w2_ref, b2_ref,
                wq_ref, bq_ref, wk_ref, bk_ref, wv_ref, bv_ref,
                qt_ref, ko_ref, vt_ref):
    x = x_ref[...]
    xn = _layer_norm(x, g_ref[...], b_ref[...])
    h = jnp.dot(xn, w1_ref[...], preferred_element_type=jnp.float32)
    h = h * jax.nn.sigmoid(h)
    kqv = jnp.dot(h, w2_ref[...], preferred_element_type=jnp.float32) + b2_ref[...]
    k = kqv[:, :L0]
    q = kqv[:, L0:2 * L0]
    v = kqv[:, 2 * L0:]
    qt_ref[...] = (jax.lax.dot_general(
        wq_ref[...], k, (((0,), (1,)), ((), ())),
        preferred_element_type=jnp.float32) + bq_ref[...]).astype(jnp.bfloat16)
    ko_ref[...] = (jnp.dot(q, wk_ref[...], preferred_element_type=jnp.float32)
                   + bk_ref[...]).astype(jnp.bfloat16)
    vt_ref[...] = (jax.lax.dot_general(
        wv_ref[...], v, (((0,), (1,)), ((), ())),
        preferred_element_type=jnp.float32) + bv_ref[...]).astype(jnp.bfloat16)


def _qkv_pass(x, p):
    nb = N_TOK // BT
    wspec = lambda shape: pl.BlockSpec(shape, lambda i: (0, 0))
    out_t = jax.ShapeDtypeStruct((L0, N_TOK), jnp.bfloat16)
    out_n = jax.ShapeDtypeStruct((N_TOK, L0), jnp.bfloat16)
    return pl.pallas_call(
        _qkv_kernel,
        grid=(nb,),
        in_specs=[
            pl.BlockSpec((BT, L0), lambda i: (i, 0)),
            wspec((1, L0)), wspec((1, L0)),
            wspec((L0, 4 * L0)), wspec((4 * L0, 3 * L0)), wspec((1, 3 * L0)),
            wspec((L0, L0)), wspec((L0, 1)),
            wspec((L0, L0)), wspec((1, L0)),
            wspec((L0, L0)), wspec((L0, 1)),
        ],
        out_specs=[pl.BlockSpec((L0, BT), lambda i: (0, i)),
                   pl.BlockSpec((BT, L0), lambda i: (i, 0)),
                   pl.BlockSpec((L0, BT), lambda i: (0, i))],
        out_shape=[out_t, out_n, out_t],
    )(x,
      p['kqv_ln_g'].reshape(1, L0), p['kqv_ln_b'].reshape(1, L0),
      p['kqv_w1'], p['kqv_w2'], p['kqv_b2'].reshape(1, 3 * L0),
      p['wq'] * INV_SCALE, p['bq'].reshape(L0, 1) * INV_SCALE,
      p['wk'], p['bk'].reshape(1, L0),
      p['wv'], p['bv'].reshape(L0, 1))


def _attn_kernel(chlo_ref, nch_ref,
                 qt_ref, k_ref, vt_ref, eq_ref, ea_ref, x_ref,
                 wo_ref, bo_ref, g_ref, b_ref, w1_ref, w2_ref, b2_ref,
                 *rest, final):
    if final:
        (fg_ref, fb_ref, fw1_ref, fw2_ref, fb2_ref, o_ref,
         m_s, l_s, acc_s) = rest
    else:
        (o_ref, m_s, l_s, acc_s) = rest
    i = pl.program_id(0)

    m_s[...] = jnp.full((NH, 1, BQ), NEG, jnp.float32)
    l_s[...] = jnp.zeros((NH, 1, BQ), jnp.float32)
    acc_s[...] = jnp.zeros((NH, DH, BQ), jnp.float32)

    eq = eq_ref[0, 0, :]
    ones_row = jnp.ones((1, CH), jnp.bfloat16)
    base = chlo_ref[i]

    def body(t, carry):
        start = (base + t) * CH
        ek = ea_ref[0, pl.ds(start, CH)]
        bias = jnp.where(ek[:, None] == eq[None, :], 0.0, NEG)
        for h in range(NH):
            kh = k_ref[pl.ds(start, CH), h * DH:(h + 1) * DH]
            qth = qt_ref[h * DH:(h + 1) * DH, :]
            vth = vt_ref[h * DH:(h + 1) * DH, pl.ds(start, CH)]
            st = jax.lax.dot_general(
                kh, qth, (((1,), (0,)), ((), ())),
                preferred_element_type=jnp.float32) + bias
            m_prev = m_s[h]
            m_new = jnp.maximum(m_prev, jnp.max(st, axis=0, keepdims=True))
            alpha = jnp.exp(m_prev - m_new)
            p = jnp.exp(st - m_new).astype(jnp.bfloat16)
            l_s[h] = l_s[h] * alpha + jax.lax.dot_general(
                ones_row, p, (((1,), (0,)), ((), ())),
                preferred_element_type=jnp.float32)
            acc_s[h] = acc_s[h] * alpha + jax.lax.dot_general(
                vth, p, (((1,), (0,)), ((), ())),
                preferred_element_type=jnp.float32)
            m_s[h] = m_new
        return carry

    jax.lax.fori_loop(0, nch_ref[i], body, 0)

    bf = jnp.bfloat16
    ot = jnp.concatenate(
        [acc_s[h] / l_s[h] for h in range(NH)], axis=0)
    o = ot.T
    attn = (jnp.dot(o.astype(bf), wo_ref[...].astype(bf),
                    preferred_element_type=jnp.float32)
            + bo_ref[...] + x_ref[...])
    xn = _layer_norm(attn, g_ref[...], b_ref[...])
    hh = jnp.dot(xn.astype(bf), w1_ref[...].astype(bf),
                 preferred_element_type=jnp.float32)
    hh = hh * jax.nn.sigmoid(hh)
    ff = (jnp.dot(hh.astype(bf), w2_ref[...].astype(bf),
                  preferred_element_type=jnp.float32) + b2_ref[...])
    res = ff + 2.0 * attn
    if final:
        xn2 = _layer_norm(res, fg_ref[...], fb_ref[...])
        h2 = jnp.dot(xn2, fw1_ref[...], preferred_element_type=jnp.float32)
        h2 = h2 * jax.nn.sigmoid(h2)
        o_ref[...] = (jnp.dot(h2, fw2_ref[...], preferred_element_type=jnp.float32)
                      + fb2_ref[...])
    else:
        o_ref[...] = res


def _attn_pass(qt, k, vt, eidx, x, p, ch_lo, nch, fin=None):
    final = fin is not None
    eq = eidx.reshape(NQ, 1, BQ)
    ea = eidx.reshape(1, N_TOK)

    qmap = lambda i, *_: (i, 0)
    wmap = lambda i, *_: (0, 0)

    in_specs = [
        pl.BlockSpec((L0, BQ), lambda i, *_: (0, i)),
        pl.BlockSpec((N_TOK, L0), wmap),
        pl.BlockSpec((L0, N_TOK), wmap),
        pl.BlockSpec((1, 1, BQ), lambda i, *_: (i, 0, 0)),
        pl.BlockSpec((1, N_TOK), wmap),
        pl.BlockSpec((BQ, L0), qmap),
        pl.BlockSpec((L0, L0), wmap), pl.BlockSpec((1, L0), wmap),
        pl.BlockSpec((1, L0), wmap), pl.BlockSpec((1, L0), wmap),
        pl.BlockSpec((L0, 4 * L0), wmap), pl.BlockSpec((4 * L0, L0), wmap),
        pl.BlockSpec((1, L0), wmap),
    ]
    args = [qt, k, vt, eq, ea, x,
            p['wo'], p['bo'].reshape(1, L0),
            p['ff_ln_g'].reshape(1, L0), p['ff_ln_b'].reshape(1, L0),
            p['ff_w1'], p['ff_w2'], p['ff_b2'].reshape(1, L0)]
    if final:
        in_specs += [
            pl.BlockSpec((1, L0), wmap), pl.BlockSpec((1, L0), wmap),
            pl.BlockSpec((L0, 4 * L0), wmap), pl.BlockSpec((4 * L0, 1), wmap),
            pl.BlockSpec((1, 1), wmap),
        ]
        args += [fin['ln_g'].reshape(1, L0), fin['ln_b'].reshape(1, L0),
                 fin['w1'], fin['w2'], fin['b2'].reshape(1, 1)]
        out_shape = jax.ShapeDtypeStruct((N_TOK, 1), jnp.float32)
        out_spec = pl.BlockSpec((BQ, 1), qmap)
    else:
        out_shape = jax.ShapeDtypeStruct((N_TOK, L0), jnp.float32)
        out_spec = pl.BlockSpec((BQ, L0), qmap)

    grid_spec = pltpu.PrefetchScalarGridSpec(
        num_scalar_prefetch=2,
        grid=(NQ,),
        in_specs=in_specs,
        out_specs=out_spec,
        scratch_shapes=[
            pltpu.VMEM((NH, 1, BQ), jnp.float32),
            pltpu.VMEM((NH, 1, BQ), jnp.float32),
            pltpu.VMEM((NH, DH, BQ), jnp.float32),
        ],
    )
    return pl.pallas_call(
        functools.partial(_attn_kernel, final=final),
        grid_spec=grid_spec,
        out_shape=out_shape,
    )(ch_lo, nch, *args)


def _kv_ranges(eidx):
    e_blk = eidx.reshape(NQ, BQ)
    e_lo = e_blk[:, 0]
    e_hi = e_blk[:, -1]
    starts = jnp.searchsorted(eidx, jnp.arange(N_ENS, dtype=eidx.dtype),
                              side='left').astype(jnp.int32)
    ends = jnp.searchsorted(eidx, jnp.arange(N_ENS, dtype=eidx.dtype),
                            side='right').astype(jnp.int32)
    ch_lo = starts[e_lo] // CH
    ch_hi = (ends[e_hi] - 1) // CH
    return ch_lo, ch_hi - ch_lo + 1


def kernel(features, ensemble_index, p1, p2, fin):
    ch_lo, nch = _kv_ranges(ensemble_index)
    qt1, k1, vt1 = _qkv_pass(features, p1)
    h1 = _attn_pass(qt1, k1, vt1, ensemble_index, features, p1, ch_lo, nch)
    qt2, k2, vt2 = _qkv_pass(h1, p2)
    out = _attn_pass(qt2, k2, vt2, ensemble_index, h1, p2, ch_lo, nch, fin=fin)
    return out


# speedup vs baseline: 1.2055x; 1.0065x over previous
import functools

import jax
import jax.numpy as jnp
from jax import lax
from jax.experimental import pallas as pl
import jax.experimental.pallas.tpu as pltpu
from jax.experimental.pallas import tpu_sc as plsc

L0 = 128
NH = 8
DH = L0 // NH
N_TOK = 32768
N_ENS = 16

BT = 1024
BQ = 512
CH = 1024
NQ = N_TOK // BQ
NCH = N_TOK // CH
INV_SCALE = 1.0 / (DH ** 0.5)
NEG = -1e30


def _layer_norm(x, g, b):
    m = jnp.mean(x, axis=-1, keepdims=True)
    v = jnp.mean((x - m) ** 2, axis=-1, keepdims=True)
    return (x - m) * jax.lax.rsqrt(v + 1e-5) * g + b


def _qkv_kernel(x_ref, g_ref, b_ref, w1_ref, w2_ref, b2_ref,
                wq_ref, bq_ref, wk_ref, bk_ref, wv_ref, bv_ref,
                qt_ref, ko_ref, vt_ref):
    x = x_ref[...]
    xn = _layer_norm(x, g_ref[...], b_ref[...])
    h = jnp.dot(xn, w1_ref[...], preferred_element_type=jnp.float32)
    h = h * jax.nn.sigmoid(h)
    kqv = jnp.dot(h, w2_ref[...], preferred_element_type=jnp.float32) + b2_ref[...]
    k = kqv[:, :L0]
    q = kqv[:, L0:2 * L0]
    v = kqv[:, 2 * L0:]
    qt_ref[...] = (jax.lax.dot_general(
        wq_ref[...], k, (((0,), (1,)), ((), ())),
        preferred_element_type=jnp.float32) + bq_ref[...]).astype(jnp.bfloat16)
    ko_ref[...] = (jnp.dot(q, wk_ref[...], preferred_element_type=jnp.float32)
                   + bk_ref[...]).astype(jnp.bfloat16)
    vt_ref[...] = (jax.lax.dot_general(
        wv_ref[...], v, (((0,), (1,)), ((), ())),
        preferred_element_type=jnp.float32) + bv_ref[...]).astype(jnp.bfloat16)


def _qkv_pass(x, p):
    nb = N_TOK // BT
    wspec = lambda shape: pl.BlockSpec(shape, lambda i: (0, 0))
    out_t = jax.ShapeDtypeStruct((L0, N_TOK), jnp.bfloat16)
    out_n = jax.ShapeDtypeStruct((N_TOK, L0), jnp.bfloat16)
    return pl.pallas_call(
        _qkv_kernel,
        grid=(nb,),
        in_specs=[
            pl.BlockSpec((BT, L0), lambda i: (i, 0)),
            wspec((1, L0)), wspec((1, L0)),
            wspec((L0, 4 * L0)), wspec((4 * L0, 3 * L0)), wspec((1, 3 * L0)),
            wspec((L0, L0)), wspec((L0, 1)),
            wspec((L0, L0)), wspec((1, L0)),
            wspec((L0, L0)), wspec((L0, 1)),
        ],
        out_specs=[pl.BlockSpec((L0, BT), lambda i: (0, i)),
                   pl.BlockSpec((BT, L0), lambda i: (i, 0)),
                   pl.BlockSpec((L0, BT), lambda i: (0, i))],
        out_shape=[out_t, out_n, out_t],
    )(x,
      p['kqv_ln_g'].reshape(1, L0), p['kqv_ln_b'].reshape(1, L0),
      p['kqv_w1'], p['kqv_w2'], p['kqv_b2'].reshape(1, 3 * L0),
      p['wq'] * INV_SCALE, p['bq'].reshape(L0, 1) * INV_SCALE,
      p['wk'], p['bk'].reshape(1, L0),
      p['wv'], p['bv'].reshape(L0, 1))


def _attn_kernel(chlo_ref, nch_ref,
                 qt_ref, k_ref, vt_ref, eq_ref, ea_ref, x_ref,
                 wo_ref, bo_ref, g_ref, b_ref, w1_ref, w2_ref, b2_ref,
                 *rest, final):
    if final:
        (fg_ref, fb_ref, fw1_ref, fw2_ref, fb2_ref, o_ref,
         m_s, l_s, acc_s) = rest
    else:
        (o_ref, m_s, l_s, acc_s) = rest
    i = pl.program_id(0)

    m_s[...] = jnp.full((NH, 1, BQ), NEG, jnp.float32)
    l_s[...] = jnp.zeros((NH, 1, BQ), jnp.float32)
    acc_s[...] = jnp.zeros((NH, DH, BQ), jnp.float32)

    eq = eq_ref[0, 0, :]
    ones_row = jnp.ones((1, CH), jnp.bfloat16)
    base = chlo_ref[i]

    def body(t, carry):
        start = (base + t) * CH
        ek = ea_ref[0, pl.ds(start, CH)]
        bias = jnp.where(ek[:, None] == eq[None, :], 0.0, NEG)
        for h in range(NH):
            kh = k_ref[pl.ds(start, CH), h * DH:(h + 1) * DH]
            qth = qt_ref[h * DH:(h + 1) * DH, :]
            vth = vt_ref[h * DH:(h + 1) * DH, pl.ds(start, CH)]
            st = jax.lax.dot_general(
                kh, qth, (((1,), (0,)), ((), ())),
                preferred_element_type=jnp.float32) + bias
            m_prev = m_s[h]
            m_new = jnp.maximum(m_prev, jnp.max(st, axis=0, keepdims=True))
            alpha = jnp.exp(m_prev - m_new)
            p = jnp.exp(st - m_new).astype(jnp.bfloat16)
            l_s[h] = l_s[h] * alpha + jax.lax.dot_general(
                ones_row, p, (((1,), (0,)), ((), ())),
                preferred_element_type=jnp.float32)
            acc_s[h] = acc_s[h] * alpha + jax.lax.dot_general(
                vth, p, (((1,), (0,)), ((), ())),
                preferred_element_type=jnp.float32)
            m_s[h] = m_new
        return carry

    jax.lax.fori_loop(0, nch_ref[i], body, 0)

    bf = jnp.bfloat16
    ot = jnp.concatenate(
        [acc_s[h] / l_s[h] for h in range(NH)], axis=0)
    o = ot.T
    attn = (jnp.dot(o.astype(bf), wo_ref[...].astype(bf),
                    preferred_element_type=jnp.float32)
            + bo_ref[...] + x_ref[...])
    xn = _layer_norm(attn, g_ref[...], b_ref[...])
    hh = jnp.dot(xn.astype(bf), w1_ref[...].astype(bf),
                 preferred_element_type=jnp.float32)
    hh = hh * jax.nn.sigmoid(hh)
    ff = (jnp.dot(hh.astype(bf), w2_ref[...].astype(bf),
                  preferred_element_type=jnp.float32) + b2_ref[...])
    res = ff + 2.0 * attn
    if final:
        xn2 = _layer_norm(res, fg_ref[...], fb_ref[...])
        h2 = jnp.dot(xn2, fw1_ref[...], preferred_element_type=jnp.float32)
        h2 = h2 * jax.nn.sigmoid(h2)
        o_ref[...] = (jnp.dot(h2, fw2_ref[...], preferred_element_type=jnp.float32)
                      + fb2_ref[...])
    else:
        o_ref[...] = res


def _attn_pass(qt, k, vt, eidx, x, p, ch_lo, nch, fin=None):
    final = fin is not None
    eq = eidx.reshape(NQ, 1, BQ)
    ea = eidx.reshape(1, N_TOK)

    qmap = lambda i, *_: (i, 0)
    wmap = lambda i, *_: (0, 0)

    in_specs = [
        pl.BlockSpec((L0, BQ), lambda i, *_: (0, i)),
        pl.BlockSpec((N_TOK, L0), wmap),
        pl.BlockSpec((L0, N_TOK), wmap),
        pl.BlockSpec((1, 1, BQ), lambda i, *_: (i, 0, 0)),
        pl.BlockSpec((1, N_TOK), wmap),
        pl.BlockSpec((BQ, L0), qmap),
        pl.BlockSpec((L0, L0), wmap), pl.BlockSpec((1, L0), wmap),
        pl.BlockSpec((1, L0), wmap), pl.BlockSpec((1, L0), wmap),
        pl.BlockSpec((L0, 4 * L0), wmap), pl.BlockSpec((4 * L0, L0), wmap),
        pl.BlockSpec((1, L0), wmap),
    ]
    args = [qt, k, vt, eq, ea, x,
            p['wo'], p['bo'].reshape(1, L0),
            p['ff_ln_g'].reshape(1, L0), p['ff_ln_b'].reshape(1, L0),
            p['ff_w1'], p['ff_w2'], p['ff_b2'].reshape(1, L0)]
    if final:
        in_specs += [
            pl.BlockSpec((1, L0), wmap), pl.BlockSpec((1, L0), wmap),
            pl.BlockSpec((L0, 4 * L0), wmap), pl.BlockSpec((4 * L0, 1), wmap),
            pl.BlockSpec((1, 1), wmap),
        ]
        args += [fin['ln_g'].reshape(1, L0), fin['ln_b'].reshape(1, L0),
                 fin['w1'], fin['w2'], fin['b2'].reshape(1, 1)]
        out_shape = jax.ShapeDtypeStruct((N_TOK, 1), jnp.float32)
        out_spec = pl.BlockSpec((BQ, 1), qmap)
    else:
        out_shape = jax.ShapeDtypeStruct((N_TOK, L0), jnp.float32)
        out_spec = pl.BlockSpec((BQ, L0), qmap)

    grid_spec = pltpu.PrefetchScalarGridSpec(
        num_scalar_prefetch=2,
        grid=(NQ,),
        in_specs=in_specs,
        out_specs=out_spec,
        scratch_shapes=[
            pltpu.VMEM((NH, 1, BQ), jnp.float32),
            pltpu.VMEM((NH, 1, BQ), jnp.float32),
            pltpu.VMEM((NH, DH, BQ), jnp.float32),
        ],
    )
    return pl.pallas_call(
        functools.partial(_attn_kernel, final=final),
        grid_spec=grid_spec,
        out_shape=out_shape,
    )(ch_lo, nch, *args)


CH_SHIFT = CH.bit_length() - 1


def _sc_ranges_kernel(e_hbm, chlo_hbm, nch_hbm, data_v, tab_v, out_v, sem):
    wid = lax.axis_index("s") * 2 + lax.axis_index("c")

    @pl.when(wid == 0)
    def _():
        pltpu.async_copy(e_hbm, data_v, sem).wait()
        iot = lax.iota(jnp.int32, 16)

        def search(le):
            lo = jnp.zeros((16,), jnp.int32)
            hi = jnp.full((16,), N_TOK, jnp.int32)
            for _ in range(16):
                mid = jnp.minimum((lo + hi) >> 1, N_TOK - 1)
                vals = plsc.load_gather(data_v, [mid])
                pred = (vals <= iot) if le else (vals < iot)
                lo = jnp.where(pred, mid + 1, lo)
                hi = jnp.where(pred, hi, mid)
            return lo

        tab_v[pl.ds(0, 16)] = search(False)
        tab_v[pl.ds(16, 16)] = search(True)
        for g in range(NQ // 16):
            pos = (iot + g * 16) * BQ
            e_lo = plsc.load_gather(data_v, [pos])
            e_hi = plsc.load_gather(data_v, [pos + (BQ - 1)])
            s_lo = plsc.load_gather(tab_v, [e_lo])
            s_hi = plsc.load_gather(tab_v, [e_hi + 16])
            chlo = lax.shift_right_logical(s_lo, CH_SHIFT)
            nch = lax.shift_right_logical(s_hi - 1, CH_SHIFT) - chlo + 1
            out_v[pl.ds(g * 16, 16)] = chlo
            out_v[pl.ds(NQ + g * 16, 16)] = nch
        pltpu.sync_copy(out_v.at[pl.ds(0, NQ)], chlo_hbm)
        pltpu.sync_copy(out_v.at[pl.ds(NQ, NQ)], nch_hbm)


def _kv_ranges(eidx):
    f = functools.partial(
        pl.kernel,
        out_type=[jax.ShapeDtypeStruct((NQ,), jnp.int32),
                  jax.ShapeDtypeStruct((NQ,), jnp.int32)],
        mesh=plsc.VectorSubcoreMesh(core_axis_name="c", subcore_axis_name="s"),
        scratch_types=[
            pltpu.VMEM((N_TOK,), jnp.int32),
            pltpu.VMEM((2 * N_ENS,), jnp.int32),
            pltpu.VMEM((2 * NQ,), jnp.int32),
            pltpu.SemaphoreType.DMA,
        ],
        compiler_params=pltpu.CompilerParams(needs_layout_passes=False),
    )(_sc_ranges_kernel)
    return f(eidx)


def kernel(features, ensemble_index, p1, p2, fin):
    ch_lo, nch = _kv_ranges(ensemble_index)
    qt1, k1, vt1 = _qkv_pass(features, p1)
    h1 = _attn_pass(qt1, k1, vt1, ensemble_index, features, p1, ch_lo, nch)
    qt2, k2, vt2 = _qkv_pass(h1, p2)
    out = _attn_pass(qt2, k2, vt2, ensemble_index, h1, p2, ch_lo, nch, fin=fin)
    return out
